# baseline (speedup 1.0000x reference)
import functools
import math

import jax
import jax.numpy as jnp
from jax import lax
from jax.experimental import pallas as pl
from jax.experimental.pallas import tpu as pltpu
from jax.experimental.pallas import tpu_sc as plsc

MEM = 65536
D = 1024
NQ = 512
K = 16
H = 8
DH = D // H
CHUNK = 128
NCHUNK = MEM // CHUNK
BLK = 4096
NBLK = MEM // BLK


def _a_body(q_ref, kb_ref, sims_ref, flat_ref, cm_sc):
    m = pl.program_id(0)
    q = q_ref[...]
    kb = kb_ref[...]
    s = lax.dot_general(q, kb, (((1,), (1,)), ((), ())),
                        preferred_element_type=jnp.float32)
    sims_ref[...] = s
    rows = [jnp.max(s[:, c * CHUNK:(c + 1) * CHUNK], axis=1)[None, :]
            for c in range(BLK // CHUNK)]
    cm_sc[pl.ds(m * (BLK // CHUNK), BLK // CHUNK), :] = (
        jnp.concatenate(rows, axis=0))

    @pl.when(m == NBLK - 1)
    def _select_chunks():
        c = cm_sc[...]
        ci = lax.broadcasted_iota(jnp.int32, (NCHUNK, NQ), 0)
        qrow = lax.broadcasted_iota(jnp.int32, (NQ,), 0)
        big = jnp.int32(1 << 30)
        neg = jnp.float32(-jnp.inf)
        for t in range(K):
            mx = jnp.max(c, axis=0, keepdims=True)
            sel = jnp.min(jnp.where(c == mx, ci, big), axis=0, keepdims=True)
            flat_ref[:, t] = qrow * NCHUNK + sel[0, :]
            c = jnp.where(ci == sel, neg, c)


def _sims_and_chunkmax(q2, memory_keys):
    return pl.pallas_call(
        _a_body,
        grid=(NBLK,),
        in_specs=[
            pl.BlockSpec((NQ, D), lambda m: (0, 0)),
            pl.BlockSpec((BLK, D), lambda m: (m, 0)),
        ],
        out_specs=[
            pl.BlockSpec((NQ, BLK), lambda m: (0, m)),
            pl.BlockSpec((NQ, K), lambda m: (0, 0)),
        ],
        out_shape=[
            jax.ShapeDtypeStruct((NQ, MEM), jnp.float32),
            jax.ShapeDtypeStruct((NQ, K), jnp.int32),
        ],
        scratch_shapes=[pltpu.VMEM((NCHUNK, NQ), jnp.float32)],
    )(q2, memory_keys)


def _sc_gather_rows(table, idx, row_chunk):
    b = idx.shape[0]
    d = table.shape[1]
    info = plsc.get_sparse_core_info()
    nc, ns = info.num_cores, info.num_subcores
    nw = nc * ns
    b_per_w = b // nw
    n_iter = b_per_w // row_chunk
    mesh = plsc.VectorSubcoreMesh(core_axis_name="c", subcore_axis_name="s")

    @functools.partial(
        pl.kernel,
        mesh=mesh,
        out_type=jax.ShapeDtypeStruct((b, d), jnp.float32),
        scratch_types=[
            pltpu.VMEM((b_per_w,), jnp.int32),
            pltpu.VMEM((row_chunk, d), jnp.float32),
            pltpu.SemaphoreType.DMA,
        ],
    )
    def k(table_hbm, idx_hbm, out_hbm, idx_v, rows_v, sem):
        wid = lax.axis_index("s") * nc + lax.axis_index("c")
        base = wid * b_per_w
        pltpu.sync_copy(idx_hbm.at[pl.ds(base, b_per_w)], idx_v)
        for j in range(n_iter):
            src = table_hbm.at[idx_v.at[pl.ds(j * row_chunk, row_chunk)]]
            pltpu.async_copy(src, rows_v, sem).wait()
            pltpu.sync_copy(rows_v, out_hbm.at[pl.ds(base + j * row_chunk,
                                                     row_chunk)])

    return k(table, idx)


def _sc_gather_kv(keys, values, rows):
    b = rows.shape[0]
    info = plsc.get_sparse_core_info()
    nc, ns = info.num_cores, info.num_subcores
    nw = nc * ns
    b_per_w = b // nw
    rc = 16
    n_iter = b_per_w // rc
    mesh = plsc.VectorSubcoreMesh(core_axis_name="c", subcore_axis_name="s")

    @functools.partial(
        pl.kernel,
        mesh=mesh,
        out_type=[
            jax.ShapeDtypeStruct((b, D), jnp.float32),
            jax.ShapeDtypeStruct((b, D), jnp.float32),
        ],
        scratch_types=[
            pltpu.VMEM((b_per_w,), jnp.int32),
            pltpu.VMEM((rc, D), jnp.float32),
            pltpu.VMEM((rc, D), jnp.float32),
            pltpu.VMEM((rc, D), jnp.float32),
            pltpu.VMEM((rc, D), jnp.float32),
            pltpu.SemaphoreType.DMA,
            pltpu.SemaphoreType.DMA,
            pltpu.SemaphoreType.DMA,
            pltpu.SemaphoreType.DMA,
        ],
    )
    def k(keys_hbm, vals_hbm, rows_hbm, rk_hbm, rv_hbm,
          idx_v, k0, k1, v0, v1, sk0, sk1, sv0, sv1):
        wid = lax.axis_index("s") * nc + lax.axis_index("c")
        base = wid * b_per_w
        pltpu.sync_copy(rows_hbm.at[pl.ds(base, b_per_w)], idx_v)
        kb = (k0, k1)
        vb = (v0, v1)
        sks = (sk0, sk1)
        svs = (sv0, sv1)
        cps = [None, None]
        for c in range(n_iter):
            p = c % 2
            src = idx_v.at[pl.ds(c * rc, rc)]
            ck = pltpu.async_copy(keys_hbm.at[src], kb[p], sks[p])
            cv = pltpu.async_copy(vals_hbm.at[src], vb[p], svs[p])
            if c > 0:
                pk, pv = cps[1 - p]
                pk.wait()
                pv.wait()
                o = pl.ds(base + (c - 1) * rc, rc)
                pltpu.sync_copy(kb[1 - p], rk_hbm.at[o])
                pltpu.sync_copy(vb[1 - p], rv_hbm.at[o])
            cps[p] = (ck, cv)
        p = (n_iter - 1) % 2
        pk, pv = cps[p]
        pk.wait()
        pv.wait()
        o = pl.ds(base + (n_iter - 1) * rc, rc)
        pltpu.sync_copy(kb[p], rk_hbm.at[o])
        pltpu.sync_copy(vb[p], rv_hbm.at[o])

    return k(keys, values, rows)


def _d_body(nq, qoff, cand_ref, flat_ref, out_ref):
    cand = cand_ref[...]
    flat = flat_ref[...]
    qrow = lax.broadcasted_iota(jnp.int32, (nq, K), 0) + qoff
    chunk = flat - qrow * NCHUNK
    lane = lax.broadcasted_iota(jnp.int32, (nq, CHUNK), 1)
    parts = [chunk[:, j:j + 1] * CHUNK + lane for j in range(K)]
    gidx = jnp.concatenate(parts, axis=1)
    big = jnp.int32(1 << 30)
    neg = jnp.float32(-jnp.inf)
    for t in range(K):
        mx = jnp.max(cand, axis=1, keepdims=True)
        sel = jnp.min(jnp.where(cand == mx, gidx, big), axis=1, keepdims=True)
        out_ref[:, t] = sel[:, 0]
        cand = jnp.where(gidx == sel, neg, cand)


def _topk_rows(cand, flat_chunks, qoff):
    nq = cand.shape[0]
    return pl.pallas_call(
        functools.partial(_d_body, nq, qoff),
        out_shape=jax.ShapeDtypeStruct((nq, K), jnp.int32),
    )(cand, flat_chunks)


GQ = 64
NGF = NQ // GQ


def _f_body(q_ref, rk_ref, rv_ref, wq_ref, wk_ref, wv_ref, wo_ref,
            bq_ref, bk_ref, bv_ref, bo_ref, out_ref):
    f32 = jnp.float32
    bf16 = jnp.bfloat16
    q16 = q_ref[...].astype(bf16)
    rk16 = rk_ref[...].astype(bf16)
    rv16 = rv_ref[...].astype(bf16)
    dims = (((1,), (1,)), ((), ()))
    qp = lax.dot_general(q16, wq_ref[...].astype(bf16), dims,
                         preferred_element_type=f32) + bq_ref[...]
    kp = lax.dot_general(rk16, wk_ref[...].astype(bf16), dims,
                         preferred_element_type=f32) + bk_ref[...]
    vp = lax.dot_general(rv16, wv_ref[...].astype(bf16), dims,
                         preferred_element_type=f32) + bv_ref[...]
    scale = f32(1.0 / math.sqrt(DH))
    outs = []
    for h in range(H):
        sl = slice(h * DH, (h + 1) * DH)
        qh = qp[:, sl]
        kh = kp[:, sl].reshape(GQ, K, DH)
        vh = vp[:, sl].reshape(GQ, K, DH)
        s = jnp.sum(kh * qh[:, None, :], axis=2) * scale
        s = s - jnp.max(s, axis=1, keepdims=True)
        e = jnp.exp(s)
        attn = e / jnp.sum(e, axis=1, keepdims=True)
        outs.append(jnp.sum(vh * attn[:, :, None], axis=1))
    att = jnp.concatenate(outs, axis=1).astype(bf16)
    out_ref[...] = lax.dot_general(att, wo_ref[...].astype(bf16), dims,
                                   preferred_element_type=f32) + bo_ref[...]


def _mha(q2, rk, rv, Wq, Wk, Wv, Wo, bq, bk, bv, bo):
    nq = q2.shape[0]
    wspec = pl.BlockSpec((D, D), lambda g: (0, 0))
    bspec = pl.BlockSpec((D,), lambda g: (0,))
    return pl.pallas_call(
        _f_body,
        grid=(nq // GQ,),
        in_specs=[
            pl.BlockSpec((GQ, D), lambda g: (g, 0)),
            pl.BlockSpec((GQ * K, D), lambda g: (g, 0)),
            pl.BlockSpec((GQ * K, D), lambda g: (g, 0)),
            wspec, wspec, wspec, wspec,
            bspec, bspec, bspec, bspec,
        ],
        out_specs=pl.BlockSpec((GQ, D), lambda g: (g, 0)),
        out_shape=jax.ShapeDtypeStruct((nq, D), jnp.float32),
    )(q2, rk, rv, Wq, Wk, Wv, Wo, bq, bk, bv, bo)


def kernel(queries, k, memory_keys, memory_values, Wq, Wk, Wv, bq, bk, bv,
           Wo, bo):
    B, S, d = queries.shape
    q2 = queries.reshape(NQ, D)
    sims, flat_chunks = _sims_and_chunkmax(q2, memory_keys)
    sims_chunks = sims.reshape(NQ * NCHUNK, CHUNK)
    shift = (jnp.asarray(k, jnp.int32) - K)
    HQ = NQ // 2
    halves = []
    for qoff in (0, HQ):
        fc = lax.slice(flat_chunks, (qoff, 0), (qoff + HQ, K))
        cand = _sc_gather_rows(sims_chunks, fc.reshape(-1), HQ * K // 32)
        top_idx = _topk_rows(cand.reshape(HQ, K * CHUNK), fc, qoff)
        rows = jnp.clip(top_idx + shift, 0, MEM - 1).reshape(-1)
        rk, rv = _sc_gather_kv(memory_keys, memory_values, rows)
        qh = lax.slice(q2, (qoff, 0), (qoff + HQ, D))
        halves.append(_mha(qh, rk, rv, Wq, Wk, Wv, Wo, bq, bk, bv, bo))
    out = jnp.concatenate(halves, axis=0)
    return out.reshape(B, S, d)


# speedup vs baseline: 1.0068x; 1.0068x over previous
import functools
import math

import jax
import jax.numpy as jnp
from jax import lax
from jax.experimental import pallas as pl
from jax.experimental.pallas import tpu as pltpu
from jax.experimental.pallas import tpu_sc as plsc

MEM = 65536
D = 1024
NQ = 512
K = 16
H = 8
DH = D // H
CHUNK = 128
NCHUNK = MEM // CHUNK
BLK = 4096
NBLK = MEM // BLK


def _a_body(q_ref, kb_ref, sims_ref, flat_ref, cm_sc):
    m = pl.program_id(0)
    q = q_ref[...]
    kb = kb_ref[...]
    s = lax.dot_general(q, kb, (((1,), (1,)), ((), ())),
                        preferred_element_type=jnp.float32)
    sims_ref[...] = s
    rows = [jnp.max(s[:, c * CHUNK:(c + 1) * CHUNK], axis=1)[None, :]
            for c in range(BLK // CHUNK)]
    cm_sc[pl.ds(m * (BLK // CHUNK), BLK // CHUNK), :] = (
        jnp.concatenate(rows, axis=0))

    @pl.when(m == NBLK - 1)
    def _select_chunks():
        c = cm_sc[...]
        ci = lax.broadcasted_iota(jnp.int32, (NCHUNK, NQ), 0)
        qrow = lax.broadcasted_iota(jnp.int32, (NQ,), 0)
        big = jnp.int32(1 << 30)
        neg = jnp.float32(-jnp.inf)
        for t in range(K):
            mx = jnp.max(c, axis=0, keepdims=True)
            sel = jnp.min(jnp.where(c == mx, ci, big), axis=0, keepdims=True)
            flat_ref[:, t] = qrow * NCHUNK + sel[0, :]
            c = jnp.where(ci == sel, neg, c)


def _sims_and_chunkmax(q2, memory_keys):
    return pl.pallas_call(
        _a_body,
        grid=(NBLK,),
        in_specs=[
            pl.BlockSpec((NQ, D), lambda m: (0, 0)),
            pl.BlockSpec((BLK, D), lambda m: (m, 0)),
        ],
        out_specs=[
            pl.BlockSpec((NQ, BLK), lambda m: (0, m)),
            pl.BlockSpec((NQ, K), lambda m: (0, 0)),
        ],
        out_shape=[
            jax.ShapeDtypeStruct((NQ, MEM), jnp.float32),
            jax.ShapeDtypeStruct((NQ, K), jnp.int32),
        ],
        scratch_shapes=[pltpu.VMEM((NCHUNK, NQ), jnp.float32)],
    )(q2, memory_keys)


def _sc_gather_rows(table, idx, row_chunk):
    b = idx.shape[0]
    d = table.shape[1]
    info = plsc.get_sparse_core_info()
    nc, ns = info.num_cores, info.num_subcores
    nw = nc * ns
    b_per_w = b // nw
    n_iter = b_per_w // row_chunk
    mesh = plsc.VectorSubcoreMesh(core_axis_name="c", subcore_axis_name="s")

    @functools.partial(
        pl.kernel,
        mesh=mesh,
        out_type=jax.ShapeDtypeStruct((b, d), jnp.float32),
        scratch_types=[
            pltpu.VMEM((b_per_w,), jnp.int32),
            pltpu.VMEM((row_chunk, d), jnp.float32),
            pltpu.SemaphoreType.DMA,
        ],
    )
    def k(table_hbm, idx_hbm, out_hbm, idx_v, rows_v, sem):
        wid = lax.axis_index("s") * nc + lax.axis_index("c")
        base = wid * b_per_w
        pltpu.sync_copy(idx_hbm.at[pl.ds(base, b_per_w)], idx_v)
        for j in range(n_iter):
            src = table_hbm.at[idx_v.at[pl.ds(j * row_chunk, row_chunk)]]
            pltpu.async_copy(src, rows_v, sem).wait()
            pltpu.sync_copy(rows_v, out_hbm.at[pl.ds(base + j * row_chunk,
                                                     row_chunk)])

    return k(table, idx)


def _sc_gather_kv(keys, values, rows):
    b = rows.shape[0]
    info = plsc.get_sparse_core_info()
    nc, ns = info.num_cores, info.num_subcores
    nw = nc * ns
    b_per_w = b // nw
    rc = 16
    n_iter = b_per_w // rc
    mesh = plsc.VectorSubcoreMesh(core_axis_name="c", subcore_axis_name="s")

    @functools.partial(
        pl.kernel,
        mesh=mesh,
        out_type=[
            jax.ShapeDtypeStruct((b, D), jnp.float32),
            jax.ShapeDtypeStruct((b, D), jnp.float32),
        ],
        scratch_types=[
            pltpu.VMEM((b_per_w,), jnp.int32),
            pltpu.VMEM((rc, D), jnp.float32),
            pltpu.VMEM((rc, D), jnp.float32),
            pltpu.VMEM((rc, D), jnp.float32),
            pltpu.VMEM((rc, D), jnp.float32),
            pltpu.SemaphoreType.DMA,
            pltpu.SemaphoreType.DMA,
            pltpu.SemaphoreType.DMA,
            pltpu.SemaphoreType.DMA,
        ],
    )
    def k(keys_hbm, vals_hbm, rows_hbm, rk_hbm, rv_hbm,
          idx_v, k0, k1, v0, v1, sk0, sk1, sv0, sv1):
        wid = lax.axis_index("s") * nc + lax.axis_index("c")
        base = wid * b_per_w
        pltpu.sync_copy(rows_hbm.at[pl.ds(base, b_per_w)], idx_v)
        kb = (k0, k1)
        vb = (v0, v1)
        sks = (sk0, sk1)
        svs = (sv0, sv1)
        cps = [None, None]
        for c in range(n_iter):
            p = c % 2
            src = idx_v.at[pl.ds(c * rc, rc)]
            ck = pltpu.async_copy(keys_hbm.at[src], kb[p], sks[p])
            cv = pltpu.async_copy(vals_hbm.at[src], vb[p], svs[p])
            if c > 0:
                pk, pv = cps[1 - p]
                pk.wait()
                pv.wait()
                o = pl.ds(base + (c - 1) * rc, rc)
                pltpu.sync_copy(kb[1 - p], rk_hbm.at[o])
                pltpu.sync_copy(vb[1 - p], rv_hbm.at[o])
            cps[p] = (ck, cv)
        p = (n_iter - 1) % 2
        pk, pv = cps[p]
        pk.wait()
        pv.wait()
        o = pl.ds(base + (n_iter - 1) * rc, rc)
        pltpu.sync_copy(kb[p], rk_hbm.at[o])
        pltpu.sync_copy(vb[p], rv_hbm.at[o])

    return k(keys, values, rows)


def _d_body(cand_ref, flat_ref, out_ref):
    cand = cand_ref[...]
    flat = flat_ref[...]
    qrow = lax.broadcasted_iota(jnp.int32, (NQ, K), 0)
    chunk = flat - qrow * NCHUNK
    lane = lax.broadcasted_iota(jnp.int32, (NQ, CHUNK), 1)
    parts = [chunk[:, j:j + 1] * CHUNK + lane for j in range(K)]
    gidx = jnp.concatenate(parts, axis=1)
    big = jnp.int32(1 << 30)
    neg = jnp.float32(-jnp.inf)
    for t in range(K):
        mx = jnp.max(cand, axis=1, keepdims=True)
        sel = jnp.min(jnp.where(cand == mx, gidx, big), axis=1, keepdims=True)
        out_ref[:, t] = sel[:, 0]
        cand = jnp.where(gidx == sel, neg, cand)


def _topk_rows(cand, flat_chunks):
    return pl.pallas_call(
        _d_body,
        out_shape=jax.ShapeDtypeStruct((NQ, K), jnp.int32),
    )(cand, flat_chunks)


GQ = 64
NGF = NQ // GQ


def _f_body(q_ref, rk_ref, rv_ref, wq_ref, wk_ref, wv_ref, wo_ref,
            bq_ref, bk_ref, bv_ref, bo_ref, out_ref):
    f32 = jnp.float32
    bf16 = jnp.bfloat16
    q16 = q_ref[...].astype(bf16)
    rk16 = rk_ref[...].astype(bf16)
    rv16 = rv_ref[...].astype(bf16)
    dims = (((1,), (1,)), ((), ()))
    qp = lax.dot_general(q16, wq_ref[...].astype(bf16), dims,
                         preferred_element_type=f32) + bq_ref[...]
    kp = lax.dot_general(rk16, wk_ref[...].astype(bf16), dims,
                         preferred_element_type=f32) + bk_ref[...]
    vp = lax.dot_general(rv16, wv_ref[...].astype(bf16), dims,
                         preferred_element_type=f32) + bv_ref[...]
    scale = f32(1.0 / math.sqrt(DH))
    outs = []
    for h in range(H):
        sl = slice(h * DH, (h + 1) * DH)
        qh = qp[:, sl]
        kh = kp[:, sl].reshape(GQ, K, DH)
        vh = vp[:, sl].reshape(GQ, K, DH)
        s = jnp.sum(kh * qh[:, None, :], axis=2) * scale
        s = s - jnp.max(s, axis=1, keepdims=True)
        e = jnp.exp(s)
        attn = e / jnp.sum(e, axis=1, keepdims=True)
        outs.append(jnp.sum(vh * attn[:, :, None], axis=1))
    att = jnp.concatenate(outs, axis=1).astype(bf16)
    out_ref[...] = lax.dot_general(att, wo_ref[...].astype(bf16), dims,
                                   preferred_element_type=f32) + bo_ref[...]


def _mha(q2, rk, rv, Wq, Wk, Wv, Wo, bq, bk, bv, bo):
    wspec = pl.BlockSpec((D, D), lambda g: (0, 0))
    bspec = pl.BlockSpec((D,), lambda g: (0,))
    return pl.pallas_call(
        _f_body,
        grid=(NGF,),
        in_specs=[
            pl.BlockSpec((GQ, D), lambda g: (g, 0)),
            pl.BlockSpec((GQ * K, D), lambda g: (g, 0)),
            pl.BlockSpec((GQ * K, D), lambda g: (g, 0)),
            wspec, wspec, wspec, wspec,
            bspec, bspec, bspec, bspec,
        ],
        out_specs=pl.BlockSpec((GQ, D), lambda g: (g, 0)),
        out_shape=jax.ShapeDtypeStruct((NQ, D), jnp.float32),
    )(q2, rk, rv, Wq, Wk, Wv, Wo, bq, bk, bv, bo)


def kernel(queries, k, memory_keys, memory_values, Wq, Wk, Wv, bq, bk, bv,
           Wo, bo):
    B, S, d = queries.shape
    q2 = queries.reshape(NQ, D)
    sims, flat_chunks = _sims_and_chunkmax(q2, memory_keys)
    sims_chunks = sims.reshape(NQ * NCHUNK, CHUNK)
    cand = _sc_gather_rows(sims_chunks, flat_chunks.reshape(-1), NQ * K // 32)
    top_idx = _topk_rows(cand.reshape(NQ, K * CHUNK), flat_chunks)
    shift = (jnp.asarray(k, jnp.int32) - K)
    rows = jnp.clip(top_idx + shift, 0, MEM - 1).reshape(-1)
    rk, rv = _sc_gather_kv(memory_keys, memory_values, rows)
    out = _mha(q2, rk, rv, Wq, Wk, Wv, Wo, bq, bk, bv, bo)
    return out.reshape(B, S, d)
